# baseline (speedup 1.0000x reference)
import functools

import jax
import jax.numpy as jnp
from jax import lax
from jax.experimental import pallas as pl
from jax.experimental.pallas import tpu as pltpu
from jax.experimental.pallas import tpu_sc as plsc

N_ATOMS = 524288
N_SYSTEMS = 8192
D_FEAT = 128
NGRP = N_ATOMS // 128

NHALF = 8
HALF = N_ATOMS // NHALF
HGRP = HALF // 128

NW = 32
CHUNK = HALF // NW
SUB = 2048
NSUB = CHUNK // SUB

MM_BLOCK = 16384
MM_G = MM_BLOCK // 128


def _mm_body(emb_ref, w_ref, out_ref):
    mm_t = lax.dot_general(
        w_ref[...], emb_ref[...],
        dimension_numbers=(((0,), (1,)), ((), ())),
        preferred_element_type=jnp.float32,
    )
    out_ref[...] = jnp.swapaxes(
        mm_t.reshape(8, MM_G, 128), 0, 1)


def _make_mm(h):
    goff = h * (HALF // MM_BLOCK)
    return pl.pallas_call(
        _mm_body,
        grid=(HALF // MM_BLOCK,),
        in_specs=[
            pl.BlockSpec((MM_BLOCK, D_FEAT), lambda i, _g=goff: (i + _g, 0)),
            pl.BlockSpec((D_FEAT, 8), lambda i: (0, 0)),
        ],
        out_specs=pl.BlockSpec((MM_G, 8, 128), lambda i: (i, 0, 0)),
        out_shape=jax.ShapeDtypeStruct((HGRP, 8, 128), jnp.float32),
        compiler_params=pltpu.CompilerParams(
            dimension_semantics=("arbitrary",)),
    )


_mm_calls = [_make_mm(h) for h in range(NHALF)]


def _comb_body(*refs):
    part_refs = refs[:NHALF]
    ds_ref, e0_ref, e1_ref = refs[NHALF:]
    energy = part_refs[0][...].sum(axis=0)
    for pr in part_refs[1:]:
        energy = energy + pr[...].sum(axis=0)
    ds = ds_ref[...]
    zero = jnp.zeros_like(energy)
    e0_ref[...] = jnp.where(ds == 0, energy, zero)
    e1_ref[...] = jnp.where(ds == 1, energy, zero)


_comb_call = pl.pallas_call(
    _comb_body,
    out_shape=[jax.ShapeDtypeStruct((N_SYSTEMS,), jnp.float32)] * 2,
)

_sc_mesh = plsc.VectorSubcoreMesh(core_axis_name="c", subcore_axis_name="s")


def _make_sc(h):
  if h == 0:
    out_type = [
        jax.ShapeDtypeStruct((NGRP * 512,), jnp.float32),
        jax.ShapeDtypeStruct((NGRP * 512,), jnp.float32),
        jax.ShapeDtypeStruct((NW, N_SYSTEMS), jnp.float32),
    ]
  else:
    out_type = jax.ShapeDtypeStruct((NW, N_SYSTEMS), jnp.float32)

  @functools.partial(
      pl.kernel,
      mesh=_sc_mesh,
      compiler_params=pltpu.CompilerParams(needs_layout_passes=False),
      out_type=out_type,
      scratch_types=[
          pltpu.VMEM((N_SYSTEMS,), jnp.int32),
          pltpu.VMEM((SUB + 16,), jnp.int32),
          pltpu.VMEM((SUB * 8,), jnp.float32),
          pltpu.VMEM((SUB * 4,), jnp.float32),
          pltpu.VMEM((SUB * 4,), jnp.float32),
          pltpu.VMEM((N_SYSTEMS,), jnp.float32),
      ],
  )
  def _sc_route(in8_hbm, b_hbm, ds_hbm, f0_hbm, f1_hbm, part_hbm,
                ds_v, b_v, in8_v, f0_v, f1_v, acc):
    wid = lax.axis_index("s") * 2 + lax.axis_index("c")
    iota = lax.iota(jnp.int32, 16)
    zero16 = jnp.zeros((16,), jnp.float32)
    one16 = jnp.ones((16,), jnp.float32)
    izero16 = jnp.zeros((16,), jnp.int32)
    i15_16 = jnp.full((16,), 15, jnp.int32)
    ineg16 = jnp.full((16,), -1, jnp.int32)

    def _ifull(x):
        return jnp.full((16,), x, jnp.int32)

    def zbody(i, carry):
        acc[pl.ds(i * 16, 16)] = zero16
        return carry
    lax.fori_loop(0, N_SYSTEMS // 16, zbody, 0)

    pltpu.sync_copy(ds_hbm, ds_v)

    for sc in range(NSUB):
        base = wid * CHUNK + sc * SUB
        pltpu.sync_copy(
            b_hbm.at[pl.ds(h * HALF + base, SUB)], b_v.at[pl.ds(0, SUB)])
        b_v[pl.ds(SUB, 16)] = ineg16
        pltpu.sync_copy(in8_hbm.at[pl.ds(base * 8, SUB * 8)], in8_v)

        def group(i, carry):
            src = (i // 8) * 1024 + (i % 8) * 16
            dst = (i // 8) * 512 + (i % 8) * 16
            b = b_v[pl.ds(i * 16, 16)]
            dsid = plsc.load_gather(ds_v, [b])
            m = jnp.where(dsid == izero16, one16, zero16)
            for c in range(3):
                v = in8_v[pl.ds(src + c * 128, 16)]
                f0 = v * m
                f0_v[pl.ds(dst + c * 128, 16)] = f0
                f1_v[pl.ds(dst + c * 128, 16)] = v - f0
            e = in8_v[pl.ds(src + 384, 16)]
            b_next = plsc.load_gather(b_v, [iota + _ifull(i * 16 + 1)])
            svec = plsc.cumsum(e)
            b_nx = jnp.where(iota == i15_16, ineg16, b_next)
            b2_nx = jnp.where(iota == i15_16, b, b_next)
            plsc.addupdate_scatter(acc, [b], svec, mask=b != b_nx)
            plsc.addupdate_scatter(
                acc, [b_next], zero16 - svec, mask=b != b2_nx)
            return carry
        lax.fori_loop(0, SUB // 16, group, 0)

        pltpu.sync_copy(
            f0_v, f0_hbm.at[pl.ds((h * HALF + base) * 4, SUB * 4)])
        pltpu.sync_copy(
            f1_v, f1_hbm.at[pl.ds((h * HALF + base) * 4, SUB * 4)])

    pltpu.sync_copy(acc, part_hbm.at[wid])

  return _sc_route


_sc_calls = [_make_sc(h) for h in range(NHALF)]


def kernel(node_emb, batch_full, dataset_ids, W_forces, w_energy):
    w8 = jnp.concatenate(
        [W_forces, w_energy[:, None],
         jnp.zeros((D_FEAT, 4), jnp.float32)], axis=1)
    batch = batch_full.astype(jnp.int32)
    ds = dataset_ids.astype(jnp.int32)

    def _to_n3(fb):
        blocks = fb.reshape(NGRP, 4, 128)
        return blocks[:, :3, :].transpose(0, 2, 1).reshape(N_ATOMS, 3)

    blk = _mm_calls[0](node_emb, w8)
    f0buf, f1buf, part0 = _sc_calls[0](blk.reshape(-1), batch, ds)
    f0_ref = jax.new_ref(f0buf)
    f1_ref = jax.new_ref(f1buf)
    parts = [part0]
    for h in range(1, NHALF):
        blk = _mm_calls[h](node_emb, w8)
        parts.append(
            _sc_calls[h](blk.reshape(-1), batch, ds, f0_ref, f1_ref))

    e0, e1 = _comb_call(*parts, ds)
    return (e0, _to_n3(f0_ref[...]), e1, _to_n3(f1_ref[...]))


# speedup vs baseline: 1.2951x; 1.2951x over previous
import functools

import jax
import jax.numpy as jnp
from jax import lax
from jax.experimental import pallas as pl
from jax.experimental.pallas import tpu as pltpu
from jax.experimental.pallas import tpu_sc as plsc

N_ATOMS = 524288
N_SYSTEMS = 8192
D_FEAT = 128
NGRP = N_ATOMS // 128

NHALF = 4
HALF = N_ATOMS // NHALF
HGRP = HALF // 128

NW = 32
CHUNK = HALF // NW
SUB = 4096
NSUB = CHUNK // SUB

MM_BLOCK = 16384
MM_G = MM_BLOCK // 128


def _mm_body(emb_ref, w_ref, out_ref):
    mm_t = lax.dot_general(
        w_ref[...], emb_ref[...],
        dimension_numbers=(((0,), (1,)), ((), ())),
        preferred_element_type=jnp.float32,
    )
    out_ref[...] = jnp.swapaxes(
        mm_t.reshape(8, MM_G, 128), 0, 1)


def _make_mm(h):
    goff = h * (HALF // MM_BLOCK)
    return pl.pallas_call(
        _mm_body,
        grid=(HALF // MM_BLOCK,),
        in_specs=[
            pl.BlockSpec((MM_BLOCK, D_FEAT), lambda i, _g=goff: (i + _g, 0)),
            pl.BlockSpec((D_FEAT, 8), lambda i: (0, 0)),
        ],
        out_specs=pl.BlockSpec((MM_G, 8, 128), lambda i: (i, 0, 0)),
        out_shape=jax.ShapeDtypeStruct((HGRP, 8, 128), jnp.float32),
        compiler_params=pltpu.CompilerParams(
            dimension_semantics=("arbitrary",)),
    )


_mm_calls = [_make_mm(h) for h in range(NHALF)]


def _comb_body(*refs):
    part_refs = refs[:NHALF]
    ds_ref, e0_ref, e1_ref = refs[NHALF:]
    energy = part_refs[0][...].sum(axis=0)
    for pr in part_refs[1:]:
        energy = energy + pr[...].sum(axis=0)
    ds = ds_ref[...]
    zero = jnp.zeros_like(energy)
    e0_ref[...] = jnp.where(ds == 0, energy, zero)
    e1_ref[...] = jnp.where(ds == 1, energy, zero)


_comb_call = pl.pallas_call(
    _comb_body,
    out_shape=[jax.ShapeDtypeStruct((N_SYSTEMS,), jnp.float32)] * 2,
)

_sc_mesh = plsc.VectorSubcoreMesh(core_axis_name="c", subcore_axis_name="s")


def _make_sc(h):
  if h == 0:
    out_type = [
        jax.ShapeDtypeStruct((NGRP * 512,), jnp.float32),
        jax.ShapeDtypeStruct((NGRP * 512,), jnp.float32),
        jax.ShapeDtypeStruct((NW, N_SYSTEMS), jnp.float32),
    ]
  else:
    out_type = jax.ShapeDtypeStruct((NW, N_SYSTEMS), jnp.float32)

  @functools.partial(
      pl.kernel,
      mesh=_sc_mesh,
      compiler_params=pltpu.CompilerParams(needs_layout_passes=False),
      out_type=out_type,
      scratch_types=[
          pltpu.VMEM((N_SYSTEMS,), jnp.int32),
          pltpu.VMEM((SUB + 16,), jnp.int32),
          pltpu.VMEM((SUB * 8,), jnp.float32),
          pltpu.VMEM((SUB * 4,), jnp.float32),
          pltpu.VMEM((SUB * 4,), jnp.float32),
          pltpu.VMEM((N_SYSTEMS,), jnp.float32),
      ],
  )
  def _sc_route(in8_hbm, b_hbm, ds_hbm, f0_hbm, f1_hbm, part_hbm,
                ds_v, b_v, in8_v, f0_v, f1_v, acc):
    wid = lax.axis_index("s") * 2 + lax.axis_index("c")
    iota = lax.iota(jnp.int32, 16)
    zero16 = jnp.zeros((16,), jnp.float32)
    one16 = jnp.ones((16,), jnp.float32)
    izero16 = jnp.zeros((16,), jnp.int32)
    i15_16 = jnp.full((16,), 15, jnp.int32)
    ineg16 = jnp.full((16,), -1, jnp.int32)

    def _ifull(x):
        return jnp.full((16,), x, jnp.int32)

    def zbody(i, carry):
        acc[pl.ds(i * 16, 16)] = zero16
        return carry
    lax.fori_loop(0, N_SYSTEMS // 16, zbody, 0)

    pltpu.sync_copy(ds_hbm, ds_v)

    for sc in range(NSUB):
        base = wid * CHUNK + sc * SUB
        pltpu.sync_copy(
            b_hbm.at[pl.ds(h * HALF + base, SUB)], b_v.at[pl.ds(0, SUB)])
        b_v[pl.ds(SUB, 16)] = ineg16
        pltpu.sync_copy(in8_hbm.at[pl.ds(base * 8, SUB * 8)], in8_v)

        @plsc.parallel_loop(0, SUB // 16, unroll=4)
        def _floop(i):
            src = (i // 8) * 1024 + (i % 8) * 16
            dst = (i // 8) * 512 + (i % 8) * 16
            b = b_v[pl.ds(i * 16, 16)]
            dsid = plsc.load_gather(ds_v, [b])
            m = jnp.where(dsid == izero16, one16, zero16)
            for c in range(3):
                v = in8_v[pl.ds(src + c * 128, 16)]
                f0 = v * m
                f0_v[pl.ds(dst + c * 128, 16)] = f0
                f1_v[pl.ds(dst + c * 128, 16)] = v - f0

        def eloop(i, carry):
            for k in range(2):
                j = i * 2 + k
                src = (j // 8) * 1024 + (j % 8) * 16
                e = in8_v[pl.ds(src + 384, 16)]
                b = b_v[pl.ds(j * 16, 16)]
                b_next = plsc.load_gather(b_v, [iota + _ifull(j * 16 + 1)])
                svec = plsc.cumsum(e)
                b_nx = jnp.where(iota == i15_16, ineg16, b_next)
                b2_nx = jnp.where(iota == i15_16, b, b_next)
                plsc.addupdate_scatter(acc, [b], svec, mask=b != b_nx)
                plsc.addupdate_scatter(
                    acc, [b_next], zero16 - svec, mask=b != b2_nx)
            return carry
        lax.fori_loop(0, SUB // 32, eloop, 0)

        pltpu.sync_copy(
            f0_v, f0_hbm.at[pl.ds((h * HALF + base) * 4, SUB * 4)])
        pltpu.sync_copy(
            f1_v, f1_hbm.at[pl.ds((h * HALF + base) * 4, SUB * 4)])

    pltpu.sync_copy(acc, part_hbm.at[wid])

  return _sc_route


_sc_calls = [_make_sc(h) for h in range(NHALF)]


def kernel(node_emb, batch_full, dataset_ids, W_forces, w_energy):
    w8 = jnp.concatenate(
        [W_forces, w_energy[:, None],
         jnp.zeros((D_FEAT, 4), jnp.float32)], axis=1)
    batch = batch_full.astype(jnp.int32)
    ds = dataset_ids.astype(jnp.int32)

    def _to_n3(fb):
        blocks = fb.reshape(NGRP, 4, 128)
        return blocks[:, :3, :].transpose(0, 2, 1).reshape(N_ATOMS, 3)

    blk = _mm_calls[0](node_emb, w8)
    f0buf, f1buf, part0 = _sc_calls[0](blk.reshape(-1), batch, ds)
    f0_ref = jax.new_ref(f0buf)
    f1_ref = jax.new_ref(f1buf)
    parts = [part0]
    for h in range(1, NHALF):
        blk = _mm_calls[h](node_emb, w8)
        parts.append(
            _sc_calls[h](blk.reshape(-1), batch, ds, f0_ref, f1_ref))

    e0, e1 = _comb_call(*parts, ds)
    return (e0, _to_n3(f0_ref[...]), e1, _to_n3(f1_ref[...]))


# speedup vs baseline: 1.3090x; 1.0108x over previous
import functools

import jax
import jax.numpy as jnp
from jax import lax
from jax.experimental import pallas as pl
from jax.experimental.pallas import tpu as pltpu
from jax.experimental.pallas import tpu_sc as plsc

N_ATOMS = 524288
N_SYSTEMS = 8192
D_FEAT = 128
NGRP = N_ATOMS // 128

NW = 32
MM_BLOCK = 16384
MM_G = MM_BLOCK // 128

CHUNK_ATOMS = [147456, 147456, 147456, 81920]
CHUNK_OFF = [0, 147456, 294912, 442368]
NHALF = len(CHUNK_ATOMS)


def _mm_body(emb_ref, w_ref, out_ref):
    mm_t = lax.dot_general(
        w_ref[...], emb_ref[...],
        dimension_numbers=(((0,), (1,)), ((), ())),
        preferred_element_type=jnp.float32,
    )
    out_ref[...] = jnp.swapaxes(
        mm_t.reshape(8, MM_G, 128), 0, 1)


def _make_mm(h):
    goff = CHUNK_OFF[h] // MM_BLOCK
    return pl.pallas_call(
        _mm_body,
        grid=(CHUNK_ATOMS[h] // MM_BLOCK,),
        in_specs=[
            pl.BlockSpec((MM_BLOCK, D_FEAT), lambda i, _g=goff: (i + _g, 0)),
            pl.BlockSpec((D_FEAT, 8), lambda i: (0, 0)),
        ],
        out_specs=pl.BlockSpec((MM_G, 8, 128), lambda i: (i, 0, 0)),
        out_shape=jax.ShapeDtypeStruct(
            (CHUNK_ATOMS[h] // 128, 8, 128), jnp.float32),
        compiler_params=pltpu.CompilerParams(
            dimension_semantics=("arbitrary",)),
    )


_mm_calls = [_make_mm(h) for h in range(NHALF)]


def _comb_body(*refs):
    part_refs = refs[:NHALF]
    ds_ref, e0_ref, e1_ref = refs[NHALF:]
    energy = part_refs[0][...].sum(axis=0)
    for pr in part_refs[1:]:
        energy = energy + pr[...].sum(axis=0)
    ds = ds_ref[...]
    zero = jnp.zeros_like(energy)
    e0_ref[...] = jnp.where(ds == 0, energy, zero)
    e1_ref[...] = jnp.where(ds == 1, energy, zero)


_comb_call = pl.pallas_call(
    _comb_body,
    out_shape=[jax.ShapeDtypeStruct((N_SYSTEMS,), jnp.float32)] * 2,
)

_sc_mesh = plsc.VectorSubcoreMesh(core_axis_name="c", subcore_axis_name="s")


def _make_sc(h):
  if h == 0:
    out_type = [
        jax.ShapeDtypeStruct((NGRP * 512,), jnp.float32),
        jax.ShapeDtypeStruct((NGRP * 512,), jnp.float32),
        jax.ShapeDtypeStruct((NW, N_SYSTEMS), jnp.float32),
    ]
  else:
    out_type = jax.ShapeDtypeStruct((NW, N_SYSTEMS), jnp.float32)

  sub = CHUNK_ATOMS[h] // NW
  off = CHUNK_OFF[h]

  @functools.partial(
      pl.kernel,
      mesh=_sc_mesh,
      compiler_params=pltpu.CompilerParams(needs_layout_passes=False),
      out_type=out_type,
      scratch_types=[
          pltpu.VMEM((N_SYSTEMS,), jnp.int32),
          pltpu.VMEM((sub + 16,), jnp.int32),
          pltpu.VMEM((sub * 8,), jnp.float32),
          pltpu.VMEM((sub * 4,), jnp.float32),
          pltpu.VMEM((sub * 4,), jnp.float32),
          pltpu.VMEM((N_SYSTEMS,), jnp.float32),
      ],
  )
  def _sc_route(in8_hbm, b_hbm, ds_hbm, f0_hbm, f1_hbm, part_hbm,
                ds_v, b_v, in8_v, f0_v, f1_v, acc):
    wid = lax.axis_index("s") * 2 + lax.axis_index("c")
    iota = lax.iota(jnp.int32, 16)
    zero16 = jnp.zeros((16,), jnp.float32)
    one16 = jnp.ones((16,), jnp.float32)
    izero16 = jnp.zeros((16,), jnp.int32)
    i15_16 = jnp.full((16,), 15, jnp.int32)
    ineg16 = jnp.full((16,), -1, jnp.int32)

    def _ifull(x):
        return jnp.full((16,), x, jnp.int32)

    def zbody(i, carry):
        acc[pl.ds(i * 16, 16)] = zero16
        return carry
    lax.fori_loop(0, N_SYSTEMS // 16, zbody, 0)

    pltpu.sync_copy(ds_hbm, ds_v)

    if True:
        base = wid * sub
        pltpu.sync_copy(
            b_hbm.at[pl.ds(off + base, sub)], b_v.at[pl.ds(0, sub)])
        b_v[pl.ds(sub, 16)] = ineg16
        pltpu.sync_copy(in8_hbm.at[pl.ds(base * 8, sub * 8)], in8_v)

        @plsc.parallel_loop(0, sub // 16, unroll=4)
        def _floop(i):
            src = (i // 8) * 1024 + (i % 8) * 16
            dst = (i // 8) * 512 + (i % 8) * 16
            b = b_v[pl.ds(i * 16, 16)]
            dsid = plsc.load_gather(ds_v, [b])
            m = jnp.where(dsid == izero16, one16, zero16)
            for c in range(3):
                v = in8_v[pl.ds(src + c * 128, 16)]
                f0 = v * m
                f0_v[pl.ds(dst + c * 128, 16)] = f0
                f1_v[pl.ds(dst + c * 128, 16)] = v - f0

        def eloop(i, carry):
            for k in range(2):
                j = i * 2 + k
                src = (j // 8) * 1024 + (j % 8) * 16
                e = in8_v[pl.ds(src + 384, 16)]
                b = b_v[pl.ds(j * 16, 16)]
                b_next = plsc.load_gather(b_v, [iota + _ifull(j * 16 + 1)])
                svec = plsc.cumsum(e)
                b_nx = jnp.where(iota == i15_16, ineg16, b_next)
                b2_nx = jnp.where(iota == i15_16, b, b_next)
                plsc.addupdate_scatter(acc, [b], svec, mask=b != b_nx)
                plsc.addupdate_scatter(
                    acc, [b_next], zero16 - svec, mask=b != b2_nx)
            return carry
        lax.fori_loop(0, sub // 32, eloop, 0)

        pltpu.sync_copy(
            f0_v, f0_hbm.at[pl.ds((off + base) * 4, sub * 4)])
        pltpu.sync_copy(
            f1_v, f1_hbm.at[pl.ds((off + base) * 4, sub * 4)])

    pltpu.sync_copy(acc, part_hbm.at[wid])

  return _sc_route


_sc_calls = [_make_sc(h) for h in range(NHALF)]


def kernel(node_emb, batch_full, dataset_ids, W_forces, w_energy):
    w8 = jnp.concatenate(
        [W_forces, w_energy[:, None],
         jnp.zeros((D_FEAT, 4), jnp.float32)], axis=1)
    batch = batch_full.astype(jnp.int32)
    ds = dataset_ids.astype(jnp.int32)

    def _to_n3(fb):
        blocks = fb.reshape(NGRP, 4, 128)
        return blocks[:, :3, :].transpose(0, 2, 1).reshape(N_ATOMS, 3)

    blk = _mm_calls[0](node_emb, w8)
    f0buf, f1buf, part0 = _sc_calls[0](blk.reshape(-1), batch, ds)
    f0_ref = jax.new_ref(f0buf)
    f1_ref = jax.new_ref(f1buf)
    parts = [part0]
    for h in range(1, NHALF):
        blk = _mm_calls[h](node_emb, w8)
        parts.append(
            _sc_calls[h](blk.reshape(-1), batch, ds, f0_ref, f1_ref))

    e0, e1 = _comb_call(*parts, ds)
    return (e0, _to_n3(f0_ref[...]), e1, _to_n3(f1_ref[...]))


# speedup vs baseline: 1.3190x; 1.0076x over previous
import functools

import jax
import jax.numpy as jnp
from jax import lax
from jax.experimental import pallas as pl
from jax.experimental.pallas import tpu as pltpu
from jax.experimental.pallas import tpu_sc as plsc

N_ATOMS = 524288
N_SYSTEMS = 8192
D_FEAT = 128
NGRP = N_ATOMS // 128

NW = 32
MM_BLOCK = 16384
MM_G = MM_BLOCK // 128

CHUNK_ATOMS = [147456, 147456, 147456, 81920]
CHUNK_OFF = [0, 147456, 294912, 442368]
NHALF = len(CHUNK_ATOMS)


def _mm_body(emb_ref, w_ref, out_ref):
    mm_t = lax.dot_general(
        w_ref[...], emb_ref[...],
        dimension_numbers=(((0,), (1,)), ((), ())),
        preferred_element_type=jnp.float32,
    )
    out_ref[...] = jnp.swapaxes(
        mm_t.reshape(8, MM_G, 128), 0, 1)


def _make_mm(h):
    goff = CHUNK_OFF[h] // MM_BLOCK
    return pl.pallas_call(
        _mm_body,
        grid=(CHUNK_ATOMS[h] // MM_BLOCK,),
        in_specs=[
            pl.BlockSpec((MM_BLOCK, D_FEAT), lambda i, _g=goff: (i + _g, 0)),
            pl.BlockSpec((D_FEAT, 8), lambda i: (0, 0)),
        ],
        out_specs=pl.BlockSpec((MM_G, 8, 128), lambda i: (i, 0, 0)),
        out_shape=jax.ShapeDtypeStruct(
            (CHUNK_ATOMS[h] // 128, 8, 128), jnp.float32),
        compiler_params=pltpu.CompilerParams(
            dimension_semantics=("arbitrary",)),
    )


_mm_calls = [_make_mm(h) for h in range(NHALF)]


def _comb_body(*refs):
    part_refs = refs[:NHALF]
    ds_ref, e0_ref, e1_ref = refs[NHALF:]
    energy = part_refs[0][...].sum(axis=0)
    for pr in part_refs[1:]:
        energy = energy + pr[...].sum(axis=0)
    ds = ds_ref[...]
    zero = jnp.zeros_like(energy)
    e0_ref[...] = jnp.where(ds == 0, energy, zero)
    e1_ref[...] = jnp.where(ds == 1, energy, zero)


_comb_call = pl.pallas_call(
    _comb_body,
    out_shape=[jax.ShapeDtypeStruct((N_SYSTEMS,), jnp.float32)] * 2,
)

_sc_mesh = plsc.VectorSubcoreMesh(core_axis_name="c", subcore_axis_name="s")


def _make_sc(h, mode="both"):
  do_f = mode in ("both", "forces")
  do_e = mode in ("both", "energy")
  out_type = []
  if h == 0:
    out_type += [
        jax.ShapeDtypeStruct((NGRP * 512,), jnp.float32),
        jax.ShapeDtypeStruct((NGRP * 512,), jnp.float32),
    ]
  if mode == "forces":
    out_type += [jax.ShapeDtypeStruct((16,), jnp.float32)]
  if do_e:
    out_type += [jax.ShapeDtypeStruct((NW, N_SYSTEMS), jnp.float32)]

  if len(out_type) == 1:
    out_type = out_type[0]

  sub = CHUNK_ATOMS[h] // NW
  off = CHUNK_OFF[h]

  scratch_types = []
  if do_f:
    scratch_types.append(pltpu.VMEM((N_SYSTEMS,), jnp.int32))
  scratch_types.append(pltpu.VMEM((sub + 16,), jnp.int32))
  scratch_types.append(pltpu.VMEM((sub * 8,), jnp.float32))
  if do_f:
    scratch_types.append(pltpu.VMEM((sub * 4,), jnp.float32))
    scratch_types.append(pltpu.VMEM((sub * 4,), jnp.float32))
  if do_e:
    scratch_types.append(pltpu.VMEM((N_SYSTEMS,), jnp.float32))
  if mode == "forces":
    scratch_types.append(pltpu.VMEM((16,), jnp.float32))

  @functools.partial(
      pl.kernel,
      mesh=_sc_mesh,
      compiler_params=pltpu.CompilerParams(needs_layout_passes=False),
      out_type=out_type,
      scratch_types=scratch_types,
  )
  def _sc_route(*refs):
    it = list(refs)
    in8_hbm, b_hbm = it[0], it[1]
    i = 2
    ds_hbm = f0_hbm = f1_hbm = tok_hbm = part_hbm = None
    if do_f:
        ds_hbm, f0_hbm, f1_hbm = it[i], it[i + 1], it[i + 2]
        i += 3
    if mode == "energy":
        i += 1
    if mode == "forces":
        tok_hbm = it[i]
        i += 1
    if do_e:
        part_hbm = it[i]
        i += 1
    scr = it[i:]
    j = 0
    ds_v = f0_v = f1_v = acc = None
    if do_f:
        ds_v = scr[j]
        j += 1
    b_v = scr[j]
    in8_v = scr[j + 1]
    j += 2
    if do_f:
        f0_v, f1_v = scr[j], scr[j + 1]
        j += 2
    if do_e:
        acc = scr[j]
        j += 1
    tok_v = scr[j] if mode == "forces" else None

    wid = lax.axis_index("s") * 2 + lax.axis_index("c")
    iota = lax.iota(jnp.int32, 16)
    zero16 = jnp.zeros((16,), jnp.float32)
    one16 = jnp.ones((16,), jnp.float32)
    izero16 = jnp.zeros((16,), jnp.int32)
    i15_16 = jnp.full((16,), 15, jnp.int32)
    ineg16 = jnp.full((16,), -1, jnp.int32)

    def _ifull(x):
        return jnp.full((16,), x, jnp.int32)

    if do_e:
        def zbody(i, carry):
            acc[pl.ds(i * 16, 16)] = zero16
            return carry
        lax.fori_loop(0, N_SYSTEMS // 16, zbody, 0)

    if do_f:
        pltpu.sync_copy(ds_hbm, ds_v)

    base = wid * sub
    pltpu.sync_copy(
        b_hbm.at[pl.ds(off + base, sub)], b_v.at[pl.ds(0, sub)])
    b_v[pl.ds(sub, 16)] = ineg16
    pltpu.sync_copy(in8_hbm.at[pl.ds(base * 8, sub * 8)], in8_v)

    if do_f:
        @plsc.parallel_loop(0, sub // 16, unroll=4)
        def _floop(i):
            src = (i // 8) * 1024 + (i % 8) * 16
            dst = (i // 8) * 512 + (i % 8) * 16
            b = b_v[pl.ds(i * 16, 16)]
            dsid = plsc.load_gather(ds_v, [b])
            m = jnp.where(dsid == izero16, one16, zero16)
            for c in range(3):
                v = in8_v[pl.ds(src + c * 128, 16)]
                f0 = v * m
                f0_v[pl.ds(dst + c * 128, 16)] = f0
                f1_v[pl.ds(dst + c * 128, 16)] = v - f0

        pltpu.sync_copy(
            f0_v, f0_hbm.at[pl.ds((off + base) * 4, sub * 4)])
        pltpu.sync_copy(
            f1_v, f1_hbm.at[pl.ds((off + base) * 4, sub * 4)])

    if do_e:
        def eloop(i, carry):
            for k in range(2):
                j = i * 2 + k
                src = (j // 8) * 1024 + (j % 8) * 16
                e = in8_v[pl.ds(src + 384, 16)]
                b = b_v[pl.ds(j * 16, 16)]
                b_next = plsc.load_gather(b_v, [iota + _ifull(j * 16 + 1)])
                svec = plsc.cumsum(e)
                b_nx = jnp.where(iota == i15_16, ineg16, b_next)
                b2_nx = jnp.where(iota == i15_16, b, b_next)
                plsc.addupdate_scatter(acc, [b], svec, mask=b != b_nx)
                plsc.addupdate_scatter(
                    acc, [b_next], zero16 - svec, mask=b != b2_nx)
            return carry
        lax.fori_loop(0, sub // 32, eloop, 0)

        pltpu.sync_copy(acc, part_hbm.at[wid])

    if mode == "forces":
        tok_v[...] = zero16
        pltpu.sync_copy(tok_v, tok_hbm)

  return _sc_route


_sc_calls = [_make_sc(h) for h in range(NHALF - 1)]
_sc_f_last = _make_sc(NHALF - 1, "forces")
_sc_e_last = _make_sc(NHALF - 1, "energy")


def kernel(node_emb, batch_full, dataset_ids, W_forces, w_energy):
    w8 = jnp.concatenate(
        [W_forces, w_energy[:, None],
         jnp.zeros((D_FEAT, 4), jnp.float32)], axis=1)
    batch = batch_full.astype(jnp.int32)
    ds = dataset_ids.astype(jnp.int32)

    def _to_n3(fb):
        blocks = fb.reshape(NGRP, 4, 128)
        return blocks[:, :3, :].transpose(0, 2, 1).reshape(N_ATOMS, 3)

    blk = _mm_calls[0](node_emb, w8)
    f0buf, f1buf, part0 = _sc_calls[0](blk.reshape(-1), batch, ds)
    f0_ref = jax.new_ref(f0buf)
    f1_ref = jax.new_ref(f1buf)
    parts = [part0]
    for h in range(1, NHALF - 1):
        blk = _mm_calls[h](node_emb, w8)
        parts.append(
            _sc_calls[h](blk.reshape(-1), batch, ds, f0_ref, f1_ref))

    blk = _mm_calls[NHALF - 1](node_emb, w8)
    in8f = blk.reshape(-1)
    tok = _sc_f_last(in8f, batch, ds, f0_ref, f1_ref)
    parts.append(_sc_e_last(in8f, batch, tok))

    e0, e1 = _comb_call(*parts, ds)
    return (e0, _to_n3(f0_ref[...]), e1, _to_n3(f1_ref[...]))
